# baseline (speedup 1.0000x reference)
import functools

import jax
import jax.numpy as jnp
from jax import lax
from jax.experimental import pallas as pl
from jax.experimental.pallas import tpu as pltpu
from jax.experimental.pallas import tpu_sc as plsc

N_NODES = 10000
E_REAL = 160000
D_IN = 1028
D_OUT = 256

G = 128
E_PAD = 163840
NW = 32
EW1 = E_PAD // NW
CH1 = EW1 // G
NSUB = 16
CH2 = E_PAD // G
NP = 10240
NHALF = 5120
CB = 16

NB = 1000


def _mm_body(x_ref, wl_ref, wr_ref, xl_ref, xr_ref):
    xb = x_ref[...]
    xl_ref[...] = jnp.dot(xb, wl_ref[...], preferred_element_type=jnp.float32)
    xr_ref[...] = jnp.dot(xb, wr_ref[...], preferred_element_type=jnp.float32)


_matmul = pl.pallas_call(
    _mm_body,
    grid=(N_NODES // NB,),
    in_specs=[
        pl.BlockSpec((NB, D_IN), lambda i: (i, 0)),
        pl.BlockSpec((D_IN, D_OUT), lambda i: (0, 0)),
        pl.BlockSpec((D_IN, D_OUT), lambda i: (0, 0)),
    ],
    out_specs=[
        pl.BlockSpec((NB, D_OUT), lambda i: (i, 0)),
        pl.BlockSpec((NB, D_OUT), lambda i: (i, 0)),
    ],
    out_shape=[
        jax.ShapeDtypeStruct((N_NODES, D_OUT), jnp.float32),
        jax.ShapeDtypeStruct((N_NODES, D_OUT), jnp.float32),
    ],
)

_mesh = plsc.VectorSubcoreMesh(
    core_axis_name="c", subcore_axis_name="s", num_cores=2, num_subcores=NSUB
)

_SC_PARAMS = pltpu.CompilerParams(needs_layout_passes=False)
_SC_PARAMS_NT = pltpu.CompilerParams(
    needs_layout_passes=False, use_tc_tiling_on_sc=False
)


@functools.partial(
    pl.kernel,
    mesh=_mesh,
    compiler_params=_SC_PARAMS,
    out_type=[
        jax.ShapeDtypeStruct((E_PAD,), jnp.float32),
        jax.ShapeDtypeStruct((NW * NP,), jnp.float32),
    ],
    scratch_types=[
        pltpu.VMEM((G,), jnp.int32),
        pltpu.VMEM((G,), jnp.int32),
        pltpu.VMEM((G, D_OUT), jnp.float32),
        pltpu.VMEM((G, D_OUT), jnp.float32),
        pltpu.VMEM((D_OUT,), jnp.float32),
        pltpu.VMEM((G,), jnp.float32),
        pltpu.VMEM((NP,), jnp.float32),
        pltpu.SemaphoreType.DMA,
        pltpu.SemaphoreType.DMA,
    ],
)
def _phase1(xl_hbm, xr_hbm, src_hbm, dst_hbm, att_hbm, p_hbm, den_hbm,
            srcb, dstb, xlb, xrb, attb, pb, denb, sem1, sem2):
    wid = lax.axis_index("s") * 2 + lax.axis_index("c")
    base = wid * EW1
    pltpu.sync_copy(att_hbm, attb)
    lanes = lax.iota(jnp.int32, 16)
    lane0 = lanes == 0
    zv = jnp.zeros((16,), jnp.float32)

    def zrow(k, carry):
        denb[pl.ds(k * 16, 16)] = zv
        return carry

    lax.fori_loop(0, NP // 16, zrow, 0)

    def chunk(i, carry):
        eb = base + i * G
        pltpu.sync_copy(src_hbm.at[pl.ds(eb, G)], srcb)
        pltpu.sync_copy(dst_hbm.at[pl.ds(eb, G)], dstb)
        cl = pltpu.async_copy(xl_hbm.at[srcb], xlb, sem1)
        cr = pltpu.async_copy(xr_hbm.at[dstb], xrb, sem2)
        cl.wait()
        cr.wait()
        for sub in range(8):
            rows = sub * 16 + lanes

            def col(j, acc):
                js = jnp.full((16,), j, jnp.int32)
                h = plsc.load_gather(xlb, [rows, js]) + plsc.load_gather(xrb, [rows, js])
                h = jnp.maximum(h, 0.2 * h)
                return acc + plsc.load_gather(attb, [js]) * h

            acc = lax.fori_loop(0, D_OUT, col, jnp.zeros((16,), jnp.float32))
            gid = eb + sub * 16 + lanes
            pb[pl.ds(sub * 16, 16)] = jnp.where(gid < E_REAL, jnp.exp(acc), 0.0)

        def dacc(e, carry2):
            dvec = plsc.load_gather(dstb, [jnp.full((16,), e, jnp.int32)])
            pvec = plsc.load_gather(pb, [jnp.full((16,), e, jnp.int32)])
            plsc.addupdate_scatter(denb, [dvec], pvec, mask=lane0)
            return carry2

        lax.fori_loop(0, G, dacc, 0)
        pltpu.sync_copy(pb, p_hbm.at[pl.ds(eb, G)])
        return carry

    lax.fori_loop(0, CH1, chunk, 0)
    pltpu.sync_copy(denb, den_hbm.at[pl.ds(wid * NP, NP)])


@functools.partial(
    pl.kernel,
    mesh=_mesh,
    compiler_params=_SC_PARAMS_NT,
    out_type=jax.ShapeDtypeStruct((NW * NHALF * CB,), jnp.float32),
    scratch_types=[
        pltpu.VMEM((G,), jnp.int32),
        pltpu.VMEM((G,), jnp.int32),
        pltpu.VMEM((G,), jnp.int32),
        pltpu.VMEM((G, CB), jnp.float32),
        pltpu.VMEM((G,), jnp.float32),
        pltpu.VMEM((NHALF * CB,), jnp.float32),
        pltpu.SemaphoreType.DMA,
    ],
)
def _phase2(xcb_hbm, src_hbm, dst_hbm, p_hbm, un_hbm,
            srcb, idxb, dstb, rowsb, pb, acc, sem):
    wid = lax.axis_index("s") * 2 + lax.axis_index("c")
    cb = wid % CB
    half = wid // CB
    lo = half * (N_NODES // 2)
    lanes = lax.iota(jnp.int32, 16)
    zv = jnp.zeros((16,), jnp.float32)

    def zrow(k, carry):
        acc[pl.ds(k * 16, 16)] = zv
        return carry

    lax.fori_loop(0, NHALF * CB // 16, zrow, 0)
    toff = cb * N_NODES

    def chunk(i, carry):
        eb = i * G
        pltpu.sync_copy(src_hbm.at[pl.ds(eb, G)], srcb)
        pltpu.sync_copy(dst_hbm.at[pl.ds(eb, G)], dstb)
        pltpu.sync_copy(p_hbm.at[pl.ds(eb, G)], pb)
        for b in range(8):
            idxb[pl.ds(b * 16, 16)] = srcb[pl.ds(b * 16, 16)] + toff
        pltpu.async_copy(xcb_hbm.at[idxb], rowsb, sem).wait()

        def edge(k, carry2):
            ks = jnp.full((16,), k, jnp.int32)
            dvec = plsc.load_gather(dstb, [ks])
            pvec = plsc.load_gather(pb, [ks])
            ok = (dvec >= lo) & (dvec < lo + N_NODES // 2)
            dl = jnp.clip(dvec - lo, 0, NHALF - 1)
            v = rowsb[k, pl.ds(0, 16)] * jnp.where(ok, pvec, 0.0)
            plsc.addupdate_scatter(acc, [dl * CB + lanes], v)
            return carry2

        lax.fori_loop(0, G, edge, 0)
        return carry

    lax.fori_loop(0, CH2, chunk, 0)
    pltpu.sync_copy(acc, un_hbm.at[pl.ds(wid * (NHALF * CB), NHALF * CB)])


def _densum_body(d_ref, o_ref):
    o_ref[...] = jnp.sum(d_ref[...], axis=0, keepdims=True)


_densum = pl.pallas_call(
    _densum_body,
    in_specs=[pl.BlockSpec((NW, NP), lambda: (0, 0))],
    out_specs=pl.BlockSpec((1, NP), lambda: (0, 0)),
    out_shape=jax.ShapeDtypeStruct((1, NP), jnp.float32),
)


def _norm_body(un_ref, den_ref, bias_ref, o_ref):
    d = den_ref[...] + 1e-16
    o_ref[...] = un_ref[...] / d + bias_ref[...]


_normalize = pl.pallas_call(
    _norm_body,
    grid=(N_NODES // NB,),
    in_specs=[
        pl.BlockSpec((NB, D_OUT), lambda i: (i, 0)),
        pl.BlockSpec((NB, 1), lambda i: (i, 0)),
        pl.BlockSpec((1, D_OUT), lambda i: (0, 0)),
    ],
    out_specs=pl.BlockSpec((NB, D_OUT), lambda i: (i, 0)),
    out_shape=jax.ShapeDtypeStruct((N_NODES, D_OUT), jnp.float32),
)


def kernel(x, edge_index, W_l, W_r, att, bias):
    src = edge_index[0]
    dst = edge_index[1]
    pad = E_PAD - E_REAL
    src_p = jnp.concatenate([src, jnp.zeros((pad,), jnp.int32)])
    dst_p = jnp.concatenate([dst, jnp.zeros((pad,), jnp.int32)])

    xl, xr = _matmul(x, W_l, W_r)
    xcb = jnp.transpose(xl.reshape(N_NODES, CB, CB), (1, 0, 2)).reshape(CB * N_NODES, CB)

    p, den_parts = _phase1(xl, xr, src_p, dst_p, att)
    un_parts = _phase2(xcb, src_p, dst_p, p)

    den = _densum(den_parts.reshape(NW, NP))
    den_col = den.reshape(NP)[:N_NODES].reshape(N_NODES, 1)

    un4 = un_parts.reshape(2, CB, NHALF, CB)[:, :, : N_NODES // 2, :]
    un2d = jnp.transpose(un4, (0, 2, 1, 3)).reshape(N_NODES, D_OUT)

    return _normalize(un2d, den_col, bias.reshape(1, D_OUT))


# speedup vs baseline: 1.4257x; 1.4257x over previous
import functools

import jax
import jax.numpy as jnp
from jax import lax
from jax.experimental import pallas as pl
from jax.experimental.pallas import tpu as pltpu
from jax.experimental.pallas import tpu_sc as plsc

N_NODES = 10000
E_REAL = 160000
D_IN = 1028
D_OUT = 256

G = 128
E_PAD = 163840
NW = 32
EW1 = E_PAD // NW
CH1 = EW1 // G
NSUB = 16
CH2 = E_PAD // G
NP = 10240
NHALF = 5120
CB = 16

NB = 1000


def _mm_body(x_ref, wl_ref, wr_ref, xl_ref, xr_ref):
    xb = x_ref[...]
    xl_ref[...] = jnp.dot(xb, wl_ref[...], preferred_element_type=jnp.float32)
    xr_ref[...] = jnp.dot(xb, wr_ref[...], preferred_element_type=jnp.float32)


_matmul = pl.pallas_call(
    _mm_body,
    grid=(N_NODES // NB,),
    in_specs=[
        pl.BlockSpec((NB, D_IN), lambda i: (i, 0)),
        pl.BlockSpec((D_IN, D_OUT), lambda i: (0, 0)),
        pl.BlockSpec((D_IN, D_OUT), lambda i: (0, 0)),
    ],
    out_specs=[
        pl.BlockSpec((NB, D_OUT), lambda i: (i, 0)),
        pl.BlockSpec((NB, D_OUT), lambda i: (i, 0)),
    ],
    out_shape=[
        jax.ShapeDtypeStruct((N_NODES, D_OUT), jnp.float32),
        jax.ShapeDtypeStruct((N_NODES, D_OUT), jnp.float32),
    ],
)

_mesh = plsc.VectorSubcoreMesh(
    core_axis_name="c", subcore_axis_name="s", num_cores=2, num_subcores=NSUB
)

_SC_PARAMS = pltpu.CompilerParams(needs_layout_passes=False)
_SC_PARAMS_NT = pltpu.CompilerParams(
    needs_layout_passes=False, use_tc_tiling_on_sc=False
)


@functools.partial(
    pl.kernel,
    mesh=_mesh,
    compiler_params=_SC_PARAMS,
    out_type=[
        jax.ShapeDtypeStruct((E_PAD,), jnp.float32),
        jax.ShapeDtypeStruct((NW * NP,), jnp.float32),
    ],
    scratch_types=[
        pltpu.VMEM((G,), jnp.int32),
        pltpu.VMEM((G,), jnp.int32),
        pltpu.VMEM((G, D_OUT), jnp.float32),
        pltpu.VMEM((G, D_OUT), jnp.float32),
        pltpu.VMEM((D_OUT,), jnp.float32),
        pltpu.VMEM((G,), jnp.float32),
        pltpu.VMEM((NP,), jnp.float32),
        pltpu.SemaphoreType.DMA,
        pltpu.SemaphoreType.DMA,
    ],
)
def _phase1(xl_hbm, xr_hbm, src_hbm, dst_hbm, att_hbm, p_hbm, den_hbm,
            srcb, dstb, xlb, xrb, attb, pb, denb, sem1, sem2):
    wid = lax.axis_index("s") * 2 + lax.axis_index("c")
    base = wid * EW1
    pltpu.sync_copy(att_hbm, attb)
    lanes = lax.iota(jnp.int32, 16)
    lane0 = lanes == 0
    zv = jnp.zeros((16,), jnp.float32)

    def zrow(k, carry):
        denb[pl.ds(k * 16, 16)] = zv
        return carry

    lax.fori_loop(0, NP // 16, zrow, 0)

    def chunk(i, carry):
        eb = base + i * G
        pltpu.sync_copy(src_hbm.at[pl.ds(eb, G)], srcb)
        pltpu.sync_copy(dst_hbm.at[pl.ds(eb, G)], dstb)
        cl = pltpu.async_copy(xl_hbm.at[srcb], xlb, sem1)
        cr = pltpu.async_copy(xr_hbm.at[dstb], xrb, sem2)
        cl.wait()
        cr.wait()
        for sub in range(8):
            rows = sub * 16 + lanes

            def col(j, acc):
                js = jnp.full((16,), j, jnp.int32)
                h = plsc.load_gather(xlb, [rows, js]) + plsc.load_gather(xrb, [rows, js])
                h = jnp.maximum(h, 0.2 * h)
                return acc + plsc.load_gather(attb, [js]) * h

            acc = lax.fori_loop(0, D_OUT, col, jnp.zeros((16,), jnp.float32))
            gid = eb + sub * 16 + lanes
            pb[pl.ds(sub * 16, 16)] = jnp.where(gid < E_REAL, jnp.exp(acc), 0.0)

        def dacc(e, carry2):
            dvec = plsc.load_gather(dstb, [jnp.full((16,), e, jnp.int32)])
            pvec = plsc.load_gather(pb, [jnp.full((16,), e, jnp.int32)])
            plsc.addupdate_scatter(denb, [dvec], pvec, mask=lane0)
            return carry2

        lax.fori_loop(0, G, dacc, 0)
        pltpu.sync_copy(pb, p_hbm.at[pl.ds(eb, G)])
        return carry

    lax.fori_loop(0, CH1, chunk, 0)
    pltpu.sync_copy(denb, den_hbm.at[pl.ds(wid * NP, NP)])


SUP = 2048
NSUP = E_PAD // SUP
SUBG = SUP // G


@functools.partial(
    pl.kernel,
    mesh=_mesh,
    compiler_params=_SC_PARAMS_NT,
    out_type=jax.ShapeDtypeStruct((NW * NHALF * CB,), jnp.float32),
    scratch_types=[
        pltpu.VMEM((SUP,), jnp.int32),
        pltpu.VMEM((SUP,), jnp.int32),
        pltpu.VMEM((SUP,), jnp.int32),
        pltpu.VMEM((SUP,), jnp.int32),
        pltpu.VMEM((SUP,), jnp.float32),
        pltpu.VMEM((SUP,), jnp.float32),
        pltpu.VMEM((G,), jnp.int32),
        pltpu.VMEM((G, CB), jnp.float32),
        pltpu.VMEM((NHALF * CB,), jnp.float32),
        pltpu.SemaphoreType.DMA,
        pltpu.SemaphoreType.DMA,
    ],
)
def _phase2(xcb_hbm, src_hbm, dst_hbm, p_hbm, un_hbm,
            srcb, idxb, dstb, dlb, pb, pmb, idxg, rowsa, acc, sema, semb):
    wid = lax.axis_index("s") * 2 + lax.axis_index("c")
    cb = wid % CB
    half = wid // CB
    lo = half * (N_NODES // 2)
    hi = lo + N_NODES // 2
    lanes = lax.iota(jnp.int32, 16)
    zv = jnp.zeros((16,), jnp.float32)

    def zrow(k, carry):
        acc[pl.ds(k * 16, 16)] = zv
        return carry

    lax.fori_loop(0, NHALF * CB // 16, zrow, 0)
    toff = cb * N_NODES

    def chunk(i, carry):
        eb = i * SUP
        pltpu.sync_copy(src_hbm.at[pl.ds(eb, SUP)], srcb)
        pltpu.sync_copy(dst_hbm.at[pl.ds(eb, SUP)], dstb)
        pltpu.sync_copy(p_hbm.at[pl.ds(eb, SUP)], pb)

        def prep(t, c2):
            sl = pl.ds(t * 16, 16)
            dv = dstb[sl]
            ok = (dv >= lo) & (dv < hi)
            pmb[sl] = jnp.where(ok, pb[sl], 0.0)
            dlb[sl] = jnp.clip(dv - lo, 0, NHALF - 1) * CB
            idxb[sl] = srcb[sl] + toff
            return c2

        lax.fori_loop(0, SUP // 16, prep, 0)

        for g in range(SUBG):
            gbase = g * G
            for b in range(G // 16):
                idxg[pl.ds(b * 16, 16)] = idxb[pl.ds(gbase + b * 16, 16)]
            pltpu.async_copy(xcb_hbm.at[idxg], rowsa, sema).wait()

            def edge(k, carry2):
                for u in range(4):
                    ks = jnp.full((16,), k * 4 + u + gbase, jnp.int32)
                    dlvec = plsc.load_gather(dlb, [ks])
                    pvec = plsc.load_gather(pmb, [ks])
                    v = rowsa[k * 4 + u, pl.ds(0, 16)] * pvec
                    plsc.addupdate_scatter(acc, [dlvec + lanes], v)
                return carry2

            lax.fori_loop(0, G // 4, edge, 0)
        return carry

    lax.fori_loop(0, NSUP, chunk, 0)
    pltpu.sync_copy(acc, un_hbm.at[pl.ds(wid * (NHALF * CB), NHALF * CB)])


def _densum_body(d_ref, o_ref):
    o_ref[...] = jnp.sum(d_ref[...], axis=0, keepdims=True)


_densum = pl.pallas_call(
    _densum_body,
    in_specs=[pl.BlockSpec((NW, NP), lambda: (0, 0))],
    out_specs=pl.BlockSpec((1, NP), lambda: (0, 0)),
    out_shape=jax.ShapeDtypeStruct((1, NP), jnp.float32),
)


def _norm_body(un_ref, den_ref, bias_ref, o_ref):
    d = den_ref[...] + 1e-16
    o_ref[...] = un_ref[...] / d + bias_ref[...]


_normalize = pl.pallas_call(
    _norm_body,
    grid=(N_NODES // NB,),
    in_specs=[
        pl.BlockSpec((NB, D_OUT), lambda i: (i, 0)),
        pl.BlockSpec((NB, 1), lambda i: (i, 0)),
        pl.BlockSpec((1, D_OUT), lambda i: (0, 0)),
    ],
    out_specs=pl.BlockSpec((NB, D_OUT), lambda i: (i, 0)),
    out_shape=jax.ShapeDtypeStruct((N_NODES, D_OUT), jnp.float32),
)


def kernel(x, edge_index, W_l, W_r, att, bias):
    src = edge_index[0]
    dst = edge_index[1]
    pad = E_PAD - E_REAL
    src_p = jnp.concatenate([src, jnp.zeros((pad,), jnp.int32)])
    dst_p = jnp.concatenate([dst, jnp.zeros((pad,), jnp.int32)])

    xl, xr = _matmul(x, W_l, W_r)
    xcb = jnp.transpose(xl.reshape(N_NODES, CB, CB), (1, 0, 2)).reshape(CB * N_NODES, CB)

    p, den_parts = _phase1(xl, xr, src_p, dst_p, att)
    un_parts = _phase2(xcb, src_p, dst_p, p)

    den = _densum(den_parts.reshape(NW, NP))
    den_col = den.reshape(NP)[:N_NODES].reshape(N_NODES, 1)

    un4 = un_parts.reshape(2, CB, NHALF, CB)[:, :, : N_NODES // 2, :]
    un2d = jnp.transpose(un4, (0, 2, 1, 3)).reshape(N_NODES, D_OUT)

    return _normalize(un2d, den_col, bias.reshape(1, D_OUT))
